# Initial kernel scaffold; baseline (speedup 1.0000x reference)
import jax
import jax.numpy as jnp
from jax.experimental import pallas as pl


def kernel(X, Y):
    raise NotImplementedError("write your pallas kernel here")


# speedup vs baseline: 8.2556x; 8.2556x over previous
import functools

import jax
import jax.numpy as jnp
from jax.experimental import pallas as pl
from jax.experimental.pallas import tpu as pltpu

BIG = 1e10


def _sdtw_kernel(x_ref, y_ref, out_ref, s_ref, *, bblk, n, d):
    nchunk = n // 128

    ones_row = jnp.ones((1, d), jnp.float32)

    def batch_body(b, carry):
        xb = x_ref[b]
        xnr = jax.lax.dot_general(
            ones_row, xb * xb, (((1,), (1,)), ((), ())),
            preferred_element_type=jnp.float32,
        )
        for qi in range(nchunk):
            yq = y_ref[b, qi * 128:(qi + 1) * 128, :]
            yn = jnp.sum(yq * yq, axis=1, keepdims=True)
            g = jax.lax.dot_general(
                yq, xb, (((1,), (1,)), ((), ())),
                preferred_element_type=jnp.float32,
            )
            d2 = yn + xnr - 2.0 * g
            s_ref[b, qi * 128:(qi + 1) * 128, :] = jnp.sqrt(jnp.maximum(d2, 0.0))
        for pj in range(nchunk):
            blk = s_ref[b, :, pj * 128:(pj + 1) * 128]
            blk = pltpu.roll(blk, pj * 128, axis=0)
            lane = jax.lax.broadcasted_iota(jnp.int32, (n, 128), 1)
            for bit in range(7):
                sh = 1 << bit
                rolled = pltpu.roll(blk, sh, axis=0)
                blk = jnp.where((lane & sh) != 0, rolled, blk)
            s_ref[b, :, pj * 128:(pj + 1) * 128] = blk
        return carry

    jax.lax.fori_loop(0, bblk, batch_body, 0)

    big = jnp.float32(BIG)
    p_iota = jax.lax.broadcasted_iota(jnp.int32, (bblk, n), 1)

    d0 = s_ref[:, 0, :]
    r1 = jnp.where(p_iota == 0, d0, big)
    r2 = jnp.full((bblk, n), big, jnp.float32)

    def diag_body(s, carry):
        r1, r2 = carry
        c = jax.lax.bitwise_and(s, n - 1)
        dvals = s_ref[:, pl.ds(c, 1), :].reshape(bblk, n)
        up = pltpu.roll(r1, 1, axis=1)
        dg = pltpu.roll(r2, 1, axis=1)
        up = jnp.where(p_iota == 0, big, up)
        dg = jnp.where(p_iota == 0, big, dg)
        lf = r1
        m = jnp.minimum(jnp.minimum(up, dg), lf)
        ssum = (jnp.exp(m - up) + jnp.exp(m - dg) + jnp.exp(m - lf))
        r_new = dvals + m - jnp.log(ssum)
        valid = (p_iota <= s) & (p_iota > s - n)
        r_new = jnp.where(valid, r_new, big)
        return (r_new, r1)

    r1, r2 = jax.lax.fori_loop(1, 2 * n - 1, diag_body, (r1, r2))
    out_ref[...] = r1[:, n - 1:n]


@jax.jit
def kernel(X, Y):
    B, N, d = X.shape
    bblk = 16
    out = pl.pallas_call(
        functools.partial(_sdtw_kernel, bblk=bblk, n=N, d=d),
        grid=(B // bblk,),
        in_specs=[
            pl.BlockSpec((bblk, N, d), lambda i: (i, 0, 0)),
            pl.BlockSpec((bblk, N, d), lambda i: (i, 0, 0)),
        ],
        out_specs=pl.BlockSpec((bblk, 1), lambda i: (i, 0)),
        out_shape=jax.ShapeDtypeStruct((B, 1), jnp.float32),
        scratch_shapes=[pltpu.VMEM((bblk, N, N), jnp.float32)],
        compiler_params=pltpu.CompilerParams(
            dimension_semantics=("parallel",),
            vmem_limit_bytes=48 * 1024 * 1024,
        ),
    )(X, Y)
    return out.reshape(B)


# speedup vs baseline: 11.1834x; 1.3546x over previous
import functools

import jax
import jax.numpy as jnp
from jax.experimental import pallas as pl
from jax.experimental.pallas import tpu as pltpu

BIG = 1e10


def _sdtw_kernel(x_hbm, y_hbm, out_ref, s_ref, xbuf, ybuf, xsem, ysem,
                 *, bblk, n, d):
    nchunk = n // 128
    gi = pl.program_id(0)

    ones_row = jnp.ones((1, d), jnp.float32)

    def copy_in(b, slot):
        gb = gi * bblk + b
        pltpu.make_async_copy(x_hbm.at[gb], xbuf.at[slot], xsem.at[slot]).start()
        pltpu.make_async_copy(y_hbm.at[gb], ybuf.at[slot], ysem.at[slot]).start()

    copy_in(0, 0)

    def batch_body(b, carry):
        slot = jax.lax.rem(b, 2)

        @pl.when(b + 1 < bblk)
        def _():
            copy_in(b + 1, jax.lax.rem(b + 1, 2))

        pltpu.make_async_copy(xbuf.at[slot], xbuf.at[slot], xsem.at[slot]).wait()
        pltpu.make_async_copy(ybuf.at[slot], ybuf.at[slot], ysem.at[slot]).wait()
        xb = xbuf[slot]
        xnr = jax.lax.dot_general(
            ones_row, xb * xb, (((1,), (1,)), ((), ())),
            preferred_element_type=jnp.float32,
        )
        for qi in range(nchunk):
            yq = ybuf[slot, qi * 128:(qi + 1) * 128, :]
            yn = jnp.sum(yq * yq, axis=1, keepdims=True)
            g = jax.lax.dot_general(
                yq, xb, (((1,), (1,)), ((), ())),
                preferred_element_type=jnp.float32,
            )
            d2 = yn + xnr - 2.0 * g
            s_ref[b, qi * 128:(qi + 1) * 128, :] = jnp.sqrt(jnp.maximum(d2, 0.0))
        for pj in range(nchunk):
            blk = s_ref[b, :, pj * 128:(pj + 1) * 128]
            blk = pltpu.roll(blk, pj * 128, axis=0)
            lane = jax.lax.broadcasted_iota(jnp.int32, (n, 128), 1)
            for bit in range(7):
                sh = 1 << bit
                rolled = pltpu.roll(blk, sh, axis=0)
                blk = jnp.where((lane & sh) != 0, rolled, blk)
            s_ref[b, :, pj * 128:(pj + 1) * 128] = blk
        return carry

    jax.lax.fori_loop(0, bblk, batch_body, 0)

    big = jnp.float32(BIG)
    p_iota = jax.lax.broadcasted_iota(jnp.int32, (bblk, n), 1)

    d0 = s_ref[:, 0, :]
    r1 = jnp.where(p_iota == 0, d0, big)
    r2 = jnp.full((bblk, n), big, jnp.float32)

    def diag_body(s, carry):
        r1, r2 = carry
        c = jax.lax.bitwise_and(s, n - 1)
        dvals = s_ref[:, pl.ds(c, 1), :].reshape(bblk, n)
        up = pltpu.roll(r1, 1, axis=1)
        dg = pltpu.roll(r2, 1, axis=1)
        up = jnp.where(p_iota == 0, big, up)
        dg = jnp.where(p_iota == 0, big, dg)
        lf = r1
        m = jnp.minimum(jnp.minimum(up, dg), lf)
        ssum = (jnp.exp(m - up) + jnp.exp(m - dg) + jnp.exp(m - lf))
        r_new = dvals + m - jnp.log(ssum)
        valid = (p_iota <= s) & (p_iota > s - n)
        r_new = jnp.where(valid, r_new, big)
        return (r_new, r1)

    r1, r2 = jax.lax.fori_loop(1, 2 * n - 1, diag_body, (r1, r2))
    out_ref[...] = r1[:, n - 1:n]


@jax.jit
def kernel(X, Y):
    B, N, d = X.shape
    bblk = 32
    out = pl.pallas_call(
        functools.partial(_sdtw_kernel, bblk=bblk, n=N, d=d),
        grid=(B // bblk,),
        in_specs=[
            pl.BlockSpec(memory_space=pl.ANY),
            pl.BlockSpec(memory_space=pl.ANY),
        ],
        out_specs=pl.BlockSpec((bblk, 1), lambda i: (i, 0)),
        out_shape=jax.ShapeDtypeStruct((B, 1), jnp.float32),
        scratch_shapes=[
            pltpu.VMEM((bblk, N, N), jnp.float32),
            pltpu.VMEM((2, N, d), jnp.float32),
            pltpu.VMEM((2, N, d), jnp.float32),
            pltpu.SemaphoreType.DMA((2,)),
            pltpu.SemaphoreType.DMA((2,)),
        ],
        compiler_params=pltpu.CompilerParams(
            dimension_semantics=("parallel",),
            vmem_limit_bytes=40 * 1024 * 1024,
        ),
    )(X, Y)
    return out.reshape(B)
